# baseline (speedup 1.0000x reference)
import functools

import jax
import jax.numpy as jnp
from jax import lax
from jax.experimental import pallas as pl
from jax.experimental.pallas import tpu as pltpu
from jax.experimental.pallas import tpu_sc as plsc

B = 16384
F = 128
C = 64
NBUF = 3


def _dot_chunk(p_ref, q_ref, tmp_ref, out_ref, out_base):
    lanes = lax.iota(jnp.int32, 16)
    perms = [lanes ^ s for s in (8, 4, 2, 1)]
    lane_eq = [lanes == l for l in range(1, 16)]

    @plsc.parallel_loop(0, C, unroll=4)
    def row(r):
        prods = [p_ref[r, pl.ds(c * 16, 16)] * q_ref[r, pl.ds(c * 16, 16)]
                 for c in range(F // 16)]
        while len(prods) > 1:
            prods = [a + b for a, b in zip(prods[::2], prods[1::2])]
        acc = prods[0]
        for perm in perms:
            acc = acc + acc.at[perm].get(mode="promise_in_bounds",
                                         unique_indices=True)
        tmp_ref[r, pl.ds(0, 16)] = acc

    @plsc.parallel_loop(0, C // 16, unroll=2)
    def compact(g):
        res = tmp_ref[g * 16, pl.ds(0, 16)]
        for l in range(1, 16):
            res = jnp.where(lane_eq[l - 1], tmp_ref[g * 16 + l, pl.ds(0, 16)],
                            res)
        out_ref[pl.ds(out_base + g * 16, 16)] = res


def kernel(u_idx, i_idx, P, Q):
    info = plsc.get_sparse_core_info()
    nc, ns = info.num_cores, info.num_subcores
    nw = nc * ns
    bpw = B // nw
    nch = bpw // C

    mesh = plsc.VectorSubcoreMesh(core_axis_name="c", subcore_axis_name="s")

    @functools.partial(
        pl.kernel,
        mesh=mesh,
        out_type=jax.ShapeDtypeStruct((B,), jnp.float32),
        scratch_types=[
            pltpu.VMEM((bpw,), jnp.int32),
            pltpu.VMEM((bpw,), jnp.int32),
            pltpu.VMEM((NBUF, C, F), jnp.float32),
            pltpu.VMEM((NBUF, C, F), jnp.float32),
            pltpu.VMEM((C, 16), jnp.float32),
            pltpu.VMEM((bpw,), jnp.float32),
            pltpu.SemaphoreType.DMA,
            pltpu.SemaphoreType.DMA,
            pltpu.SemaphoreType.DMA,
            pltpu.SemaphoreType.DMA,
        ],
    )
    def run(u_hbm, i_hbm, p_hbm, q_hbm, out_hbm, u_v, i_v, p_buf, q_buf,
            tmp_v, out_v, idx_sem, sem0, sem1, sem2):
        sems = [sem0, sem1, sem2]
        wid = lax.axis_index("s") * nc + lax.axis_index("c")
        base = wid * bpw

        cp_u0 = pltpu.async_copy(u_hbm.at[pl.ds(base, C)],
                                 u_v.at[pl.ds(0, C)], idx_sem)
        cp_i0 = pltpu.async_copy(i_hbm.at[pl.ds(base, C)],
                                 i_v.at[pl.ds(0, C)], idx_sem)
        cp_u1 = pltpu.async_copy(u_hbm.at[pl.ds(base + C, bpw - C)],
                                 u_v.at[pl.ds(C, bpw - C)], idx_sem)
        cp_i1 = pltpu.async_copy(i_hbm.at[pl.ds(base + C, bpw - C)],
                                 i_v.at[pl.ds(C, bpw - C)], idx_sem)
        cp_u0.wait()
        cp_i0.wait()

        def start_chunk(ch):
            b = ch % NBUF
            cp_p = pltpu.async_copy(
                p_hbm.at[u_v.at[pl.ds(ch * C, C)]], p_buf.at[b], sems[b])
            cp_q = pltpu.async_copy(
                q_hbm.at[i_v.at[pl.ds(ch * C, C)]], q_buf.at[b], sems[b])
            return cp_p, cp_q

        pending = [start_chunk(0), None, None]
        cp_u1.wait()
        cp_i1.wait()
        pending[1] = start_chunk(1)
        out_cps = []
        for ch in range(nch):
            cp_p, cp_q = pending[ch % NBUF]
            cp_p.wait()
            cp_q.wait()
            if ch + 2 < nch:
                pending[(ch + 2) % NBUF] = start_chunk(ch + 2)
            _dot_chunk(p_buf.at[ch % NBUF], q_buf.at[ch % NBUF], tmp_v, out_v,
                       ch * C)
            out_cps.append(pltpu.async_copy(
                out_v.at[pl.ds(ch * C, C)],
                out_hbm.at[pl.ds(base + ch * C, C)], idx_sem))

        for cp in out_cps:
            cp.wait()

    return run(u_idx, i_idx, P, Q)


# speedup vs baseline: 1.0200x; 1.0200x over previous
import functools

import jax
import jax.numpy as jnp
from jax import lax
from jax.experimental import pallas as pl
from jax.experimental.pallas import tpu as pltpu
from jax.experimental.pallas import tpu_sc as plsc

B = 16384
F = 128
C = 128
NBUF = 3


def _dot_chunk(p_ref, q_ref, tmp_ref, out_ref, out_base, n):
    lanes = lax.iota(jnp.int32, 16)
    perms = [lanes ^ s for s in (8, 4, 2, 1)]
    lane_eq = [lanes == l for l in range(1, 16)]

    @plsc.parallel_loop(0, n, unroll=4)
    def row(r):
        prods = [p_ref[r, pl.ds(c * 16, 16)] * q_ref[r, pl.ds(c * 16, 16)]
                 for c in range(F // 16)]
        while len(prods) > 1:
            prods = [a + b for a, b in zip(prods[::2], prods[1::2])]
        acc = prods[0]
        for perm in perms:
            acc = acc + acc.at[perm].get(mode="promise_in_bounds",
                                         unique_indices=True)
        tmp_ref[r, pl.ds(0, 16)] = acc

    @plsc.parallel_loop(0, n // 16, unroll=2)
    def compact(g):
        res = tmp_ref[g * 16, pl.ds(0, 16)]
        for l in range(1, 16):
            res = jnp.where(lane_eq[l - 1], tmp_ref[g * 16 + l, pl.ds(0, 16)],
                            res)
        out_ref[pl.ds(out_base + g * 16, 16)] = res


def kernel(u_idx, i_idx, P, Q):
    info = plsc.get_sparse_core_info()
    nc, ns = info.num_cores, info.num_subcores
    nw = nc * ns
    bpw = B // nw
    sizes = [C] * (bpw // C - 1) + [3 * C // 4, C // 4]
    offs = [sum(sizes[:k]) for k in range(len(sizes))]
    nch = len(sizes)

    mesh = plsc.VectorSubcoreMesh(core_axis_name="c", subcore_axis_name="s")

    @functools.partial(
        pl.kernel,
        mesh=mesh,
        out_type=jax.ShapeDtypeStruct((B,), jnp.float32),
        scratch_types=[
            pltpu.VMEM((bpw,), jnp.int32),
            pltpu.VMEM((bpw,), jnp.int32),
            pltpu.VMEM((NBUF, C, F), jnp.float32),
            pltpu.VMEM((NBUF, C, F), jnp.float32),
            pltpu.VMEM((C, 16), jnp.float32),
            pltpu.VMEM((bpw,), jnp.float32),
            pltpu.SemaphoreType.DMA,
            pltpu.SemaphoreType.DMA,
            pltpu.SemaphoreType.DMA,
            pltpu.SemaphoreType.DMA,
        ],
    )
    def run(u_hbm, i_hbm, p_hbm, q_hbm, out_hbm, u_v, i_v, p_buf, q_buf,
            tmp_v, out_v, idx_sem, sem0, sem1, sem2):
        sems = [sem0, sem1, sem2]
        wid = lax.axis_index("s") * nc + lax.axis_index("c")
        base = wid * bpw

        cp_u0 = pltpu.async_copy(u_hbm.at[pl.ds(base, C)],
                                 u_v.at[pl.ds(0, C)], idx_sem)
        cp_i0 = pltpu.async_copy(i_hbm.at[pl.ds(base, C)],
                                 i_v.at[pl.ds(0, C)], idx_sem)
        cp_u1 = pltpu.async_copy(u_hbm.at[pl.ds(base + C, bpw - C)],
                                 u_v.at[pl.ds(C, bpw - C)], idx_sem)
        cp_i1 = pltpu.async_copy(i_hbm.at[pl.ds(base + C, bpw - C)],
                                 i_v.at[pl.ds(C, bpw - C)], idx_sem)
        cp_u0.wait()
        cp_i0.wait()

        def start_chunk(ch):
            b = ch % NBUF
            off, sz = offs[ch], sizes[ch]
            cp_p = pltpu.async_copy(
                p_hbm.at[u_v.at[pl.ds(off, sz)]],
                p_buf.at[b, pl.ds(0, sz)], sems[b])
            cp_q = pltpu.async_copy(
                q_hbm.at[i_v.at[pl.ds(off, sz)]],
                q_buf.at[b, pl.ds(0, sz)], sems[b])
            return cp_p, cp_q

        pending = [start_chunk(0), None, None]
        cp_u1.wait()
        cp_i1.wait()
        pending[1] = start_chunk(1)
        out_cps = []
        for ch in range(nch):
            cp_p, cp_q = pending[ch % NBUF]
            cp_p.wait()
            cp_q.wait()
            if ch + 2 < nch:
                pending[(ch + 2) % NBUF] = start_chunk(ch + 2)
            _dot_chunk(p_buf.at[ch % NBUF], q_buf.at[ch % NBUF], tmp_v, out_v,
                       offs[ch], sizes[ch])
            out_cps.append(pltpu.async_copy(
                out_v.at[pl.ds(offs[ch], sizes[ch])],
                out_hbm.at[pl.ds(base + offs[ch], sizes[ch])], idx_sem))

        for cp in out_cps:
            cp.wait()

    return run(u_idx, i_idx, P, Q)


# speedup vs baseline: 1.0339x; 1.0136x over previous
import functools

import jax
import jax.numpy as jnp
from jax import lax
from jax.experimental import pallas as pl
from jax.experimental.pallas import tpu as pltpu
from jax.experimental.pallas import tpu_sc as plsc

B = 16384
F = 128
C = 128
NBUF = 3


def _dot_chunk(p_ref, q_ref, tmp_ref, out_ref, out_base):
    lanes = lax.iota(jnp.int32, 16)
    perms = [lanes ^ s for s in (8, 4, 2, 1)]
    lane_eq = [lanes == l for l in range(1, 16)]

    @plsc.parallel_loop(0, C, unroll=4)
    def row(r):
        prods = [p_ref[r, pl.ds(c * 16, 16)] * q_ref[r, pl.ds(c * 16, 16)]
                 for c in range(F // 16)]
        while len(prods) > 1:
            prods = [a + b for a, b in zip(prods[::2], prods[1::2])]
        acc = prods[0]
        for perm in perms:
            acc = acc + acc.at[perm].get(mode="promise_in_bounds",
                                         unique_indices=True)
        tmp_ref[r, pl.ds(0, 16)] = acc

    @plsc.parallel_loop(0, C // 16, unroll=2)
    def compact(g):
        res = tmp_ref[g * 16, pl.ds(0, 16)]
        for l in range(1, 16):
            res = jnp.where(lane_eq[l - 1], tmp_ref[g * 16 + l, pl.ds(0, 16)],
                            res)
        out_ref[pl.ds(out_base + g * 16, 16)] = res


def kernel(u_idx, i_idx, P, Q):
    info = plsc.get_sparse_core_info()
    nc, ns = info.num_cores, info.num_subcores
    nw = nc * ns
    bpw = B // nw
    nch = bpw // C

    mesh = plsc.VectorSubcoreMesh(core_axis_name="c", subcore_axis_name="s")

    @functools.partial(
        pl.kernel,
        mesh=mesh,
        out_type=jax.ShapeDtypeStruct((B,), jnp.float32),
        scratch_types=[
            pltpu.VMEM((bpw,), jnp.int32),
            pltpu.VMEM((bpw,), jnp.int32),
            pltpu.VMEM((NBUF, C, F), jnp.float32),
            pltpu.VMEM((NBUF, C, F), jnp.float32),
            pltpu.VMEM((C, 16), jnp.float32),
            pltpu.VMEM((bpw,), jnp.float32),
            pltpu.SemaphoreType.DMA,
            pltpu.SemaphoreType.DMA,
            pltpu.SemaphoreType.DMA,
            pltpu.SemaphoreType.DMA,
        ],
    )
    def run(u_hbm, i_hbm, p_hbm, q_hbm, out_hbm, u_v, i_v, p_buf, q_buf,
            tmp_v, out_v, idx_sem, sem0, sem1, sem2):
        sems = [sem0, sem1, sem2]
        wid = lax.axis_index("s") * nc + lax.axis_index("c")
        base = wid * bpw

        cp_u0 = pltpu.async_copy(u_hbm.at[pl.ds(base, C)],
                                 u_v.at[pl.ds(0, C)], idx_sem)
        cp_i0 = pltpu.async_copy(i_hbm.at[pl.ds(base, C)],
                                 i_v.at[pl.ds(0, C)], idx_sem)
        cp_u1 = pltpu.async_copy(u_hbm.at[pl.ds(base + C, bpw - C)],
                                 u_v.at[pl.ds(C, bpw - C)], idx_sem)
        cp_i1 = pltpu.async_copy(i_hbm.at[pl.ds(base + C, bpw - C)],
                                 i_v.at[pl.ds(C, bpw - C)], idx_sem)
        cp_u0.wait()
        cp_i0.wait()

        def start_chunk(ch):
            b = ch % NBUF
            cp_p = pltpu.async_copy(
                p_hbm.at[u_v.at[pl.ds(ch * C, C)]], p_buf.at[b], sems[b])
            cp_q = pltpu.async_copy(
                q_hbm.at[i_v.at[pl.ds(ch * C, C)]], q_buf.at[b], sems[b])
            return cp_p, cp_q

        pending = [start_chunk(0), None, None]
        cp_u1.wait()
        cp_i1.wait()
        pending[1] = start_chunk(1)
        out_cps = []
        for ch in range(nch):
            cp_p, cp_q = pending[ch % NBUF]
            cp_p.wait()
            cp_q.wait()
            if ch + 2 < nch:
                pending[(ch + 2) % NBUF] = start_chunk(ch + 2)
            _dot_chunk(p_buf.at[ch % NBUF], q_buf.at[ch % NBUF], tmp_v, out_v,
                       ch * C)
            out_cps.append(pltpu.async_copy(
                out_v.at[pl.ds(ch * C, C)],
                out_hbm.at[pl.ds(base + ch * C, C)], idx_sem))

        for cp in out_cps:
            cp.wait()

    return run(u_idx, i_idx, P, Q)
